# Initial kernel scaffold; baseline (speedup 1.0000x reference)
import jax
import jax.numpy as jnp
from jax.experimental import pallas as pl


def kernel(node_feats, edge_index, count_matrix, library_size, basis, alpha, proportion, W_enc0, b_enc0, W_enc_convs, W_enc2, b_enc2, W_dec0, b_dec0, W_dec_convs, W_dec2, b_dec2, W_fac, b_fac, W_alpha, b_alpha, loading, gamma):
    raise NotImplementedError("write your pallas kernel here")


# speedup vs baseline: 4.7849x; 4.7849x over previous
import functools

import numpy as np
import jax
import jax.numpy as jnp
from jax import lax
from jax.experimental import pallas as pl
from jax.experimental.pallas import tpu as pltpu, tpu_sc as plsc

N = 10000
NP = 10240
E = 320000
G = 128
H1 = 256
H2 = 128
K = 10
S = 64
NL = 8
COEF_FE = 0.1 * 9.25
COEF_REG = 0.01
BETAS = [float(np.log(1.0 / (i + 1) + 1.0)) for i in range(NL)]

NC = 2
NS = 16
CH = 128
EPS = E // NS
NCH = 157
EPAD = NCH * CH - EPS
RPT = NP // NS

BLK = 512
NBLK = NP // BLK


def _elu(x):
    return jnp.where(x > 0, x, jnp.exp(jnp.minimum(x, 0.0)) - 1.0)


def _sc_prop_body(y_hbm, idx_hbm, out_hbm, ibuf, gbuf, acc_sh, sem):
    c = lax.axis_index("c")
    s = lax.axis_index("s")
    def _zrow(i, carry):
        for j in range(G // 16):
            gbuf[i, pl.ds(j * 16, 16)] = jnp.zeros((16,), jnp.float32)
        return carry
    lax.fori_loop(0, CH, _zrow, 0)
    for t in range(RPT // CH):
        pltpu.sync_copy(gbuf, acc_sh.at[pl.ds(s * RPT + t * CH, CH)])
    plsc.subcore_barrier()

    def _chunk(j, carry):
        pltpu.sync_copy(idx_hbm.at[c, s, j], ibuf)
        pltpu.async_copy(y_hbm.at[ibuf.at[0]], gbuf, sem).wait()
        pltpu.sync_copy(gbuf, acc_sh.at[ibuf.at[1]], add=True)
        return carry
    lax.fori_loop(0, NCH, _chunk, 0)
    plsc.subcore_barrier()
    pltpu.sync_copy(acc_sh.at[pl.ds(s * RPT, RPT)],
                    out_hbm.at[c, pl.ds(s * RPT, RPT)])


@functools.cache
def _sc_prop_kernel():
    return functools.partial(
        pl.kernel,
        out_type=jax.ShapeDtypeStruct((NC, NP, G), jnp.float32),
        mesh=plsc.VectorSubcoreMesh(core_axis_name="c", subcore_axis_name="s",
                                    num_cores=NC, num_subcores=NS),
        scratch_types=[
            pltpu.VMEM((2, CH), jnp.int32),
            pltpu.VMEM((CH, G), jnp.float32),
            pltpu.VMEM_SHARED((NP, G), jnp.float32),
            pltpu.SemaphoreType.DMA,
        ],
    )(_sc_prop_body)


def _sc_prop(y, idxc):
    return _sc_prop_kernel()(y, idxc)


def _enc0_body(nf_ref, w_ref, b_ref, deg_ref, x0_ref, y_ref, dinv_ref):
    x = _elu(
        jnp.dot(nf_ref[...], w_ref[...], preferred_element_type=jnp.float32)
        + b_ref[...])
    deg = deg_ref[...]
    dinv = jnp.where(deg > 0, lax.rsqrt(jnp.maximum(deg, 1e-12)), 0.0)
    x0_ref[...] = x
    y = dinv * x
    y_ref[0, :, :] = y[:, :G]
    y_ref[1, :, :] = y[:, G:]
    dinv_ref[...] = dinv


def _tc_enc0(nf, w, b, deg):
    return pl.pallas_call(
        _enc0_body,
        grid=(NBLK,),
        in_specs=[
            pl.BlockSpec((BLK, G), lambda i: (i, 0)),
            pl.BlockSpec((G, H1), lambda i: (0, 0)),
            pl.BlockSpec((1, H1), lambda i: (0, 0)),
            pl.BlockSpec((BLK, 1), lambda i: (i, 0)),
        ],
        out_specs=[
            pl.BlockSpec((BLK, H1), lambda i: (i, 0)),
            pl.BlockSpec((2, BLK, G), lambda i: (0, i, 0)),
            pl.BlockSpec((BLK, 1), lambda i: (i, 0)),
        ],
        out_shape=[
            jax.ShapeDtypeStruct((NP, H1), jnp.float32),
            jax.ShapeDtypeStruct((2, NP, G), jnp.float32),
            jax.ShapeDtypeStruct((NP, 1), jnp.float32),
        ],
    )(nf, w, b, deg)


def _layer_body(beta_ref, agg_ref, x0_ref, dinv_ref, wc_ref, x_ref, y_ref):
    beta = beta_ref[0, 0]
    dinv = dinv_ref[...]
    h = jnp.concatenate([agg_ref[0, :, :], agg_ref[1, :, :]], axis=1) * dinv
    h = 0.9 * h + 0.1 * x0_ref[...]
    x = _elu(
        (1.0 - beta) * h
        + beta * jnp.dot(h, wc_ref[...], preferred_element_type=jnp.float32))
    x_ref[...] = x
    y = dinv * x
    y_ref[0, :, :] = y[:, :G]
    y_ref[1, :, :] = y[:, G:]


def _tc_layer(beta, agg, x0, dinv, wc):
    return pl.pallas_call(
        _layer_body,
        grid=(NBLK,),
        in_specs=[
            pl.BlockSpec((1, 1), lambda i: (0, 0), memory_space=pltpu.SMEM),
            pl.BlockSpec((2, BLK, G), lambda i: (0, i, 0)),
            pl.BlockSpec((BLK, H1), lambda i: (i, 0)),
            pl.BlockSpec((BLK, 1), lambda i: (i, 0)),
            pl.BlockSpec((H1, H1), lambda i: (0, 0)),
        ],
        out_specs=[
            pl.BlockSpec((BLK, H1), lambda i: (i, 0)),
            pl.BlockSpec((2, BLK, G), lambda i: (0, i, 0)),
        ],
        out_shape=[
            jax.ShapeDtypeStruct((NP, H1), jnp.float32),
            jax.ShapeDtypeStruct((2, NP, G), jnp.float32),
        ],
    )(beta, agg, x0, dinv, wc)


def _mid_body(x_ref, we2_ref, be2_ref, wd0_ref, bd0_ref, dinv_ref,
              xd0_ref, y_ref, ez_ref):
    z = (jnp.dot(x_ref[...], we2_ref[...], preferred_element_type=jnp.float32)
         + be2_ref[...])
    xd = _elu(
        jnp.dot(z, wd0_ref[...], preferred_element_type=jnp.float32)
        + bd0_ref[...])
    dinv = dinv_ref[...]
    xd0_ref[...] = xd
    y = dinv * xd
    y_ref[0, :, :] = y[:, :G]
    y_ref[1, :, :] = y[:, G:]
    ez_ref[...] = _elu(z)


def _tc_mid(x, we2, be2, wd0, bd0, dinv):
    return pl.pallas_call(
        _mid_body,
        grid=(NBLK,),
        in_specs=[
            pl.BlockSpec((BLK, H1), lambda i: (i, 0)),
            pl.BlockSpec((H1, H2), lambda i: (0, 0)),
            pl.BlockSpec((1, H2), lambda i: (0, 0)),
            pl.BlockSpec((H2, H1), lambda i: (0, 0)),
            pl.BlockSpec((1, H1), lambda i: (0, 0)),
            pl.BlockSpec((BLK, 1), lambda i: (i, 0)),
        ],
        out_specs=[
            pl.BlockSpec((BLK, H1), lambda i: (i, 0)),
            pl.BlockSpec((2, BLK, G), lambda i: (0, i, 0)),
            pl.BlockSpec((BLK, H2), lambda i: (i, 0)),
        ],
        out_shape=[
            jax.ShapeDtypeStruct((NP, H1), jnp.float32),
            jax.ShapeDtypeStruct((2, NP, G), jnp.float32),
            jax.ShapeDtypeStruct((NP, H2), jnp.float32),
        ],
    )(x, we2, be2, wd0, bd0, dinv)


def _final_body(xd_ref, wd2_ref, bd2_ref, nf_ref, ez_ref, wfac_ref, bfac_ref,
                wa_ref, ba_ref, load_ref, prop_ref, basis_ref, alpha_ref,
                cnt_ref, lib_ref, gamma_ref,
                fl_ref, ds_ref, f2_ref, l2_ref):
    i = pl.program_id(0)

    @pl.when(i == 0)
    def _init():
        fl_ref[0, 0] = 0.0
        ds_ref[0, 0] = 0.0
        f2_ref[0, 0] = 0.0
        l2_ref[0, 0] = jnp.sum(load_ref[...] * load_ref[...])

    mask = (lax.broadcasted_iota(jnp.int32, (BLK, 1), 0) + i * BLK) < N

    recon = (jnp.dot(xd_ref[...], wd2_ref[...],
                     preferred_element_type=jnp.float32) + bd2_ref[...])
    d = nf_ref[...] - recon
    fl = jnp.sum(jnp.where(
        mask, jnp.sqrt(jnp.sum(d * d, axis=1, keepdims=True)), 0.0))
    fl_ref[0, 0] += fl

    ez = ez_ref[...]
    u = jnp.zeros((BLK, G), jnp.float32)
    f2 = jnp.zeros((), jnp.float32)
    for k in range(K):
        fac = (jnp.dot(ez, wfac_ref[k, :, :],
                       preferred_element_type=jnp.float32)
               + bfac_ref[k, :][None, :])
        f2 += jnp.sum(jnp.where(mask, fac * fac, 0.0))
        u += (jnp.exp(jnp.dot(fac, load_ref[k, :, :],
                              preferred_element_type=jnp.float32))
              * (prop_ref[:, k:k + 1] * basis_ref[k, :][None, :]))
    alpha_res = (jnp.dot(ez, wa_ref[...],
                         preferred_element_type=jnp.float32) + ba_ref[0, 0])
    log_lam = (jnp.log(u + 1e-6) + alpha_ref[...] + alpha_res
               + gamma_ref[...])
    lib = lib_ref[...]
    term = (cnt_ref[...] * (jnp.log(lib + 1e-6) + log_lam)
            - lib * jnp.exp(log_lam))
    ds = jnp.sum(jnp.where(mask, jnp.sum(term, axis=1, keepdims=True), 0.0))
    ds_ref[0, 0] += ds
    f2_ref[0, 0] += f2


def _tc_final(xd, wd2, bd2, nf, ez, wfac, bfac, wa, ba, load, prop, basis,
              alpha, cnt, lib, gamma):
    one = lambda i: (0, 0)
    return pl.pallas_call(
        _final_body,
        grid=(NBLK,),
        in_specs=[
            pl.BlockSpec((BLK, H1), lambda i: (i, 0)),
            pl.BlockSpec((H1, G), one),
            pl.BlockSpec((1, G), one),
            pl.BlockSpec((BLK, G), lambda i: (i, 0)),
            pl.BlockSpec((BLK, H2), lambda i: (i, 0)),
            pl.BlockSpec((K, H2, S), lambda i: (0, 0, 0)),
            pl.BlockSpec((K, S), one),
            pl.BlockSpec((H2, 1), one),
            pl.BlockSpec((1, 1), one, memory_space=pltpu.SMEM),
            pl.BlockSpec((K, S, G), lambda i: (0, 0, 0)),
            pl.BlockSpec((BLK, K), lambda i: (i, 0)),
            pl.BlockSpec((K, G), one),
            pl.BlockSpec((BLK, 1), lambda i: (i, 0)),
            pl.BlockSpec((BLK, G), lambda i: (i, 0)),
            pl.BlockSpec((BLK, 1), lambda i: (i, 0)),
            pl.BlockSpec((1, G), one),
        ],
        out_specs=[pl.BlockSpec((1, 1), one, memory_space=pltpu.SMEM)] * 4,
        out_shape=[jax.ShapeDtypeStruct((1, 1), jnp.float32)] * 4,
    )(xd, wd2, bd2, nf, ez, wfac, bfac, wa, ba, load, prop, basis, alpha,
      cnt, lib, gamma)


def kernel(node_feats, edge_index, count_matrix, library_size, basis, alpha,
           proportion, W_enc0, b_enc0, W_enc_convs, W_enc2, b_enc2, W_dec0,
           b_dec0, W_dec_convs, W_dec2, b_dec2, W_fac, b_fac, W_alpha,
           b_alpha, loading, gamma):
    row = edge_index[0]
    col = edge_index[1]
    padi = jnp.full((NS, EPAD), N, jnp.int32)
    rowp = jnp.concatenate([row.reshape(NS, EPS), padi], axis=1)
    rowp = rowp.reshape(NS, NCH, CH)
    colp = jnp.concatenate([col.reshape(NS, EPS), padi], axis=1)
    colp = colp.reshape(NS, NCH, CH)
    pair = jnp.stack([rowp, colp], axis=2)
    core_off = jnp.array([NP, 0], jnp.int32)[None, None, :, None]
    idxc = jnp.stack([pair, pair + core_off])

    nf_p = jnp.pad(node_feats, ((0, NP - N), (0, 0)))
    prop_p = jnp.pad(proportion, ((0, NP - N), (0, 0)))
    alpha_p = jnp.pad(alpha, (0, NP - N)).reshape(NP, 1)
    cnt_p = jnp.pad(count_matrix, ((0, NP - N), (0, 0)))
    lib_p = jnp.pad(library_size, ((0, NP - N), (0, 0)))

    ones_y = jnp.ones((NC * NP, G), jnp.float32)
    deg = _sc_prop(ones_y, idxc)[0, :, 0:1]

    x0e, y, dinv = _tc_enc0(nf_p, W_enc0, b_enc0.reshape(1, H1), deg)
    beta_ops = [jnp.full((1, 1), b, jnp.float32) for b in BETAS]
    x = x0e
    for i in range(NL):
        agg = _sc_prop(y.reshape(NC * NP, G), idxc)
        x, y = _tc_layer(beta_ops[i], agg, x0e, dinv, W_enc_convs[i])

    xd0, y, ez = _tc_mid(x, W_enc2, b_enc2.reshape(1, H2), W_dec0,
                         b_dec0.reshape(1, H1), dinv)
    x = xd0
    for i in range(NL):
        agg = _sc_prop(y.reshape(NC * NP, G), idxc)
        x, y = _tc_layer(beta_ops[i], agg, xd0, dinv, W_dec_convs[i])

    fl, ds, f2, l2 = _tc_final(
        x, W_dec2, b_dec2.reshape(1, G), nf_p, ez, W_fac, b_fac, W_alpha,
        b_alpha.reshape(1, 1), loading, prop_p, basis, alpha_p, cnt_p, lib_p,
        gamma.reshape(1, G))

    return (-(ds[0, 0] / N)
            + COEF_FE * (fl[0, 0] / N)
            + COEF_REG * (f2[0, 0] / (N * S) + l2[0, 0] / (G * S)))
